# Initial kernel scaffold; baseline (speedup 1.0000x reference)
import jax
import jax.numpy as jnp
from jax.experimental import pallas as pl


def kernel(x, edge_index, batch_size, W, b, gamma, beta):
    raise NotImplementedError("write your pallas kernel here")


# speedup vs baseline: 8.4708x; 8.4708x over previous
import functools

import jax
import jax.numpy as jnp
from jax import lax
from jax.experimental import pallas as pl
from jax.experimental.pallas import tpu as pltpu
from jax.experimental.pallas import tpu_sc as plsc

N = 10000
E = 320000
D = 128
NC, NS = 2, 16
NW = NC * NS
EPW = E // NW
K = 80
NCHUNK = EPW // K
NPAD = 10240
RPT = NPAD // NS


def _sc_aggregate(x, row3, col3):
    mesh = plsc.VectorSubcoreMesh(core_axis_name="c", subcore_axis_name="s")

    @functools.partial(
        pl.kernel,
        out_type=(
            jax.ShapeDtypeStruct((NC, NPAD, D), jnp.float32),
            jax.ShapeDtypeStruct((NC, NPAD), jnp.float32),
        ),
        mesh=mesh,
        scratch_types=[
            pltpu.VMEM_SHARED((NPAD, D), jnp.float32),
            pltpu.VMEM_SHARED((NPAD,), jnp.float32),
            pltpu.VMEM((NCHUNK, K), jnp.int32),
            pltpu.VMEM((NCHUNK, K), jnp.int32),
            pltpu.VMEM((K, D), jnp.float32),
            pltpu.VMEM((K,), jnp.float32),
            pltpu.VMEM((RPT,), jnp.float32),
            pltpu.SemaphoreType.DMA,
        ],
    )
    def agg_kernel(x_hbm, row_hbm, col_hbm, agg_hbm, cnt_hbm,
                   agg_s, cnt_s, ridx, cidx, rows, ones, zcnt, sem):
        c = lax.axis_index("c")
        s = lax.axis_index("s")
        wid = s * NC + c

        zv = jnp.zeros((16,), jnp.float32)
        ov = jnp.ones((16,), jnp.float32)

        def _zc(i, carry):
            zcnt[pl.ds(i * 16, 16)] = zv
            return carry
        lax.fori_loop(0, RPT // 16, _zc, 0)
        for j in range(K // 16):
            ones[pl.ds(j * 16, 16)] = ov

        def _zr(i, carry):
            for j in range(D // 16):
                rows[i, pl.ds(j * 16, 16)] = zv
            return carry
        lax.fori_loop(0, K, _zr, 0)

        base_r = s * RPT
        for jb in range(RPT // K):
            pltpu.sync_copy(rows, agg_s.at[pl.ds(base_r + jb * K, K), :])
        pltpu.sync_copy(zcnt, cnt_s.at[pl.ds(base_r, RPT)])

        pltpu.sync_copy(row_hbm.at[wid], ridx)
        pltpu.sync_copy(col_hbm.at[wid], cidx)

        plsc.subcore_barrier()

        def _chunk(j, carry):
            pltpu.async_copy(x_hbm.at[cidx.at[j]], rows, sem).wait()
            pltpu.sync_copy(rows, agg_s.at[ridx.at[j]], add=True)
            pltpu.sync_copy(ones, cnt_s.at[ridx.at[j]], add=True)
            return carry
        lax.fori_loop(0, NCHUNK, _chunk, 0)

        plsc.subcore_barrier()

        pltpu.sync_copy(agg_s.at[pl.ds(base_r, RPT), :],
                        agg_hbm.at[c, pl.ds(base_r, RPT), :])
        pltpu.sync_copy(cnt_s.at[pl.ds(base_r, RPT)],
                        cnt_hbm.at[c, pl.ds(base_r, RPT)])

    return agg_kernel(x, row3, col3)


def _tc_body(x_ref, agg_ref, cnt_ref, w_ref, b_ref, g_ref, be_ref, out_ref):
    x = x_ref[...]
    agg = agg_ref[0, :N, :] + agg_ref[1, :N, :]
    cnt = cnt_ref[0, :N, :] + cnt_ref[1, :N, :]
    inv = 1.0 / (cnt + 1e-8)
    u = x + agg * inv
    y = lax.dot_general(u, w_ref[...], (((1,), (1,)), ((), ())),
                        preferred_element_type=jnp.float32)
    y = y + (1.0 + cnt * inv) * b_ref[...]
    mean = jnp.mean(y, axis=0, keepdims=True)
    yc = y - mean
    var = jnp.mean(yc * yc, axis=0, keepdims=True)
    out_ref[...] = yc * lax.rsqrt(var + 1e-5) * g_ref[...] + be_ref[...]


def kernel(x, edge_index, batch_size, W, b, gamma, beta):
    del batch_size
    ei = edge_index.astype(jnp.int32)
    row3 = ei[0].reshape(NW, NCHUNK, K)
    col3 = ei[1].reshape(NW, NCHUNK, K)
    agg, cnt = _sc_aggregate(x, row3, col3)
    cnt3 = cnt.reshape(NC, NPAD, 1)
    out = pl.pallas_call(
        _tc_body,
        out_shape=jax.ShapeDtypeStruct((N, D), jnp.float32),
    )(x, agg, cnt3, W, b.reshape(1, D), gamma.reshape(1, D),
      beta.reshape(1, D))
    return out
